# baseline (speedup 1.0000x reference)
import numpy as np
import jax
import jax.numpy as jnp
from jax.experimental import pallas as pl
from jax.experimental.pallas import tpu as pltpu

_C = 20
_A = 5
_HW = 19
_POS = _HW * _HW
_PP = 384
_NC = _A * _POS
_NP = 1824
_NW = _NP // 16
_STRIDE = 32.0
_T = 0.45
_PRE = 0.005
_B0 = np.array([1.08, 1.19, 3.42, 4.41, 6.63, 11.38, 9.42, 5.11, 16.62, 10.52],
               dtype=np.float32).reshape(5, 2)

_TILES = [(i * 128, 128) for i in range(14)] + [(14 * 128, 32)]


def _body(x_ref, im_ref, prob_ref, bb_ref, cS, rT, Mp):
    f32 = jnp.float32
    col = jax.lax.broadcasted_iota(jnp.int32, (1, _POS), 1)
    gx = (col % _HW).astype(f32)
    gy = (col // _HW).astype(f32)
    imh = im_ref[0, 0, 0]
    imw = im_ref[0, 0, 1]

    sxy = jax.nn.sigmoid(x_ref[0, 0:2 * _A, :])
    ewh = jnp.exp(x_ref[0, 2 * _A:4 * _A, :])
    sob = jax.nn.sigmoid(x_ref[0, 4 * _A:5 * _A, :])
    px1, py1, px2, py2, par, ps0 = [], [], [], [], [], []
    for a in range(_A):
        bx = (sxy[2 * a:2 * a + 1, :] + gx) * _STRIDE
        by = (sxy[2 * a + 1:2 * a + 2, :] + gy) * _STRIDE
        bw = ewh[2 * a:2 * a + 1, :] * _B0[a, 0] * _STRIDE
        bh = ewh[2 * a + 1:2 * a + 2, :] * _B0[a, 1] * _STRIDE
        x1 = (bx - bw * 0.5) / imw
        y1 = (by - bh * 0.5) / imh
        x2 = (bx + bw * 0.5) / imw
        y2 = (by + bh * 0.5) / imh
        area = jnp.maximum(x2 - x1, 0.0) * jnp.maximum(y2 - y1, 0.0)
        c0 = _C + _A + _C * a
        cblk = x_ref[0, c0:c0 + _C, :]
        m = jnp.max(cblk, axis=0, keepdims=True)
        se = jnp.sum(jnp.exp(cblk - m), axis=0, keepdims=True)
        s = (1.0 / se) * sob[a:a + 1, :]
        s0 = jnp.where(s > _PRE, s, 0.0)
        px1.append(x1)
        py1.append(y1)
        px2.append(x2)
        py2.append(y2)
        par.append(area)
        ps0.append(s0)
    z = jnp.zeros((1, _NP - _NC), f32)
    cS[0:1, :] = jnp.concatenate(px1 + [z], axis=1)
    cS[1:2, :] = jnp.concatenate(py1 + [z], axis=1)
    cS[2:3, :] = jnp.concatenate(px2 + [z], axis=1)
    cS[3:4, :] = jnp.concatenate(py2 + [z], axis=1)
    cS[4:5, :] = jnp.concatenate(par + [z], axis=1)
    s0row = jnp.concatenate(ps0 + [z], axis=1)
    cS[5:6, :] = s0row
    cS[6:7, :] = jnp.where(s0row > 0.0, s0row, -1.0)
    cS[7:8, :] = jnp.zeros((1, _NP), f32)
    for c in range(4):
        bb_ref[0, c, :] = cS[c, :]
    rT[...] = jnp.transpose(cS[...], (1, 0))

    cx1 = cS[0:1, :]
    cy1 = cS[1:2, :]
    cx2 = cS[2:3, :]
    cy2 = cS[3:4, :]
    car = cS[4:5, :]
    cs0 = cS[5:6, :]
    pr = jax.lax.broadcasted_iota(jnp.int32, (8, 128), 0)
    pc = jax.lax.broadcasted_iota(jnp.int32, (8, 128), 1)
    PW = jnp.where(pc // 16 == pr, (1 << (pc % 16)).astype(f32), 0.0).astype(jnp.bfloat16)
    rl = jax.lax.broadcasted_iota(jnp.int32, (128, 128), 0)
    cl = jax.lax.broadcasted_iota(jnp.int32, (128, 128), 1)
    s2r = jax.lax.broadcasted_iota(jnp.int32, (_NP - 128, (_NP - 128) // 16), 0)
    s2c = jax.lax.broadcasted_iota(jnp.int32, (_NP - 128, (_NP - 128) // 16), 1)
    S2 = jnp.where(s2r // 16 == s2c, (1 << (s2r % 16)).astype(f32), 0.0).astype(jnp.bfloat16)

    def iou(rows, csl):
        ix1 = jnp.maximum(rows[:, 0:1], cx1[:, csl])
        iy1 = jnp.maximum(rows[:, 1:2], cy1[:, csl])
        ix2 = jnp.minimum(rows[:, 2:3], cx2[:, csl])
        iy2 = jnp.minimum(rows[:, 3:4], cy2[:, csl])
        inter = jnp.maximum(ix2 - ix1, 0.0) * jnp.maximum(iy2 - iy1, 0.0)
        return inter / (rows[:, 4:5] + car[:, csl] - inter + 1e-9)

    for base, ni in _TILES:
        rows = rT[base:base + ni, :]
        rs0 = rows[:, 6:7]
        dsl = slice(base, base + ni)
        qd = iou(rows, dsl)
        precd = ((rs0 > cs0[:, dsl])
                 | ((rs0 == cs0[:, dsl]) & (rl[:ni, :ni] < cl[:ni, :ni])))
        Md = ((qd > _T) & precd).astype(jnp.bfloat16)
        Mp[base // 16:(base + ni) // 16, dsl] = jnp.dot(
            PW[:ni // 16, :ni], Md, preferred_element_type=f32).astype(jnp.int32)
        if base + ni == _NP:
            break
        usl = slice(base + ni, _NP)
        W = _NP - (base + ni)
        Of = (iou(rows, usl) > _T).astype(jnp.bfloat16)
        Mu = Of * (rs0 >= cs0[:, usl]).astype(jnp.bfloat16)
        Mp[base // 16:(base + ni) // 16, usl] = jnp.dot(
            PW[:ni // 16, :ni], Mu, preferred_element_type=f32).astype(jnp.int32)
        c6 = cS[6:7, usl]
        r5 = rows[:, 5:6]
        Mlt = Of * (c6 > r5).astype(jnp.bfloat16)
        pkT = jnp.dot(Mlt, S2[:W, :W // 16], preferred_element_type=f32)
        Mp[(base + ni) // 16:_NW, dsl] = jnp.transpose(pkT, (1, 0)).astype(jnp.int32)

    jr = jax.lax.broadcasted_iota(jnp.int32, (_NP, 128), 0)
    wc = jax.lax.broadcasted_iota(jnp.int32, (_NP, 128), 1)
    S = jnp.where(jr // 16 == wc, (1 << (jr % 16)).astype(f32), 0.0).astype(jnp.bfloat16)
    Mpv = Mp[...]

    def keep_of(kw):
        sup = jnp.max(Mpv & kw, axis=0, keepdims=True)
        return sup == 0

    def repack(kb):
        kwf = jnp.dot(kb.astype(jnp.bfloat16), S, preferred_element_type=f32)
        return jnp.transpose(kwf, (1, 0))[:_NW, :].astype(jnp.int32)

    def cond(c):
        return c[1]

    def step(c):
        kw, _ = c
        nkw = repack(keep_of(kw))
        return nkw, jnp.any(nkw != kw)

    kw0 = jnp.full((_NW, 1), 0xFFFF, jnp.int32)
    kwf, _ = jax.lax.while_loop(cond, step, (kw0, jnp.bool_(True)))
    prob = keep_of(kwf).astype(f32) * cs0
    prob_ref[0, 0, :] = prob[0]


def kernel(x, im_info):
    B = x.shape[0]
    xp = x.reshape(B, 125, _POS)
    prob_pad, bb_pad = pl.pallas_call(
        _body,
        grid=(B,),
        in_specs=[
            pl.BlockSpec((1, 125, _POS), lambda i: (i, 0, 0)),
            pl.BlockSpec((1, 1, 2), lambda i: (i, 0, 0)),
        ],
        out_specs=[
            pl.BlockSpec((1, 1, _NP), lambda i: (i, 0, 0)),
            pl.BlockSpec((1, 4, _NP), lambda i: (i, 0, 0)),
        ],
        out_shape=[
            jax.ShapeDtypeStruct((B, 1, _NP), jnp.float32),
            jax.ShapeDtypeStruct((B, 4, _NP), jnp.float32),
        ],
        scratch_shapes=[
            pltpu.VMEM((8, _NP), jnp.float32),
            pltpu.VMEM((_NP, 8), jnp.float32),
            pltpu.VMEM((_NW, _NP), jnp.int32),
        ],
    )(xp, im_info.reshape(B, 1, 2))
    prob = prob_pad[:, 0, :_NC]
    bboxs = jnp.moveaxis(bb_pad[:, :, :_NC], 1, 2)
    return prob, bboxs


# speedup vs baseline: 1.0070x; 1.0070x over previous
import numpy as np
import jax
import jax.numpy as jnp
from jax.experimental import pallas as pl
from jax.experimental.pallas import tpu as pltpu

_C = 20
_A = 5
_HW = 19
_POS = _HW * _HW
_PP = 384
_NC = _A * _POS
_NP = 1824
_NW = _NP // 16
_STRIDE = 32.0
_T = 0.45
_PRE = 0.005
_B0 = np.array([1.08, 1.19, 3.42, 4.41, 6.63, 11.38, 9.42, 5.11, 16.62, 10.52],
               dtype=np.float32).reshape(5, 2)

_TILES = [(i * 128, 128) for i in range(14)] + [(14 * 128, 32)]


def _body(x_ref, im_ref, prob_ref, bb_ref, cS, rT, Mp):
    f32 = jnp.float32
    col = jax.lax.broadcasted_iota(jnp.int32, (1, _POS), 1)
    gx = (col % _HW).astype(f32)
    gy = (col // _HW).astype(f32)
    imh = im_ref[0, 0, 0]
    imw = im_ref[0, 0, 1]

    sxy = jax.nn.sigmoid(x_ref[0, 0:2 * _A, :])
    ewh = jnp.exp(x_ref[0, 2 * _A:4 * _A, :])
    sob = jax.nn.sigmoid(x_ref[0, 4 * _A:5 * _A, :])
    px1, py1, px2, py2, par, ps0 = [], [], [], [], [], []
    for a in range(_A):
        bx = (sxy[2 * a:2 * a + 1, :] + gx) * _STRIDE
        by = (sxy[2 * a + 1:2 * a + 2, :] + gy) * _STRIDE
        bw = ewh[2 * a:2 * a + 1, :] * _B0[a, 0] * _STRIDE
        bh = ewh[2 * a + 1:2 * a + 2, :] * _B0[a, 1] * _STRIDE
        x1 = (bx - bw * 0.5) / imw
        y1 = (by - bh * 0.5) / imh
        x2 = (bx + bw * 0.5) / imw
        y2 = (by + bh * 0.5) / imh
        area = jnp.maximum(x2 - x1, 0.0) * jnp.maximum(y2 - y1, 0.0)
        c0 = _C + _A + _C * a
        cblk = x_ref[0, c0:c0 + _C, :]
        m = jnp.max(cblk, axis=0, keepdims=True)
        se = jnp.sum(jnp.exp(cblk - m), axis=0, keepdims=True)
        s = (1.0 / se) * sob[a:a + 1, :]
        s0 = jnp.where(s > _PRE, s, 0.0)
        px1.append(x1)
        py1.append(y1)
        px2.append(x2)
        py2.append(y2)
        par.append(area)
        ps0.append(s0)
    z = jnp.zeros((1, _NP - _NC), f32)
    cS[0:1, :] = jnp.concatenate(px1 + [z], axis=1)
    cS[1:2, :] = jnp.concatenate(py1 + [z], axis=1)
    cS[2:3, :] = jnp.concatenate(px2 + [z], axis=1)
    cS[3:4, :] = jnp.concatenate(py2 + [z], axis=1)
    cS[4:5, :] = jnp.concatenate(par + [z], axis=1)
    s0row = jnp.concatenate(ps0 + [z], axis=1)
    cS[5:6, :] = s0row
    cS[6:7, :] = jnp.where(s0row > 0.0, s0row, -1.0)
    cS[7:8, :] = jnp.zeros((1, _NP), f32)
    for c in range(4):
        bb_ref[0, c, :] = cS[c, :]
    rT[...] = jnp.transpose(cS[...], (1, 0))

    cx1 = cS[0:1, :]
    cy1 = cS[1:2, :]
    cx2 = cS[2:3, :]
    cy2 = cS[3:4, :]
    car = cS[4:5, :]
    cs0 = cS[5:6, :]
    pr = jax.lax.broadcasted_iota(jnp.int32, (8, 128), 0)
    pc = jax.lax.broadcasted_iota(jnp.int32, (8, 128), 1)
    PW = jnp.where(pc // 16 == pr, (1 << (pc % 16)).astype(f32), 0.0)
    rl = jax.lax.broadcasted_iota(jnp.int32, (128, 128), 0)
    cl = jax.lax.broadcasted_iota(jnp.int32, (128, 128), 1)
    s2r = jax.lax.broadcasted_iota(jnp.int32, (_NP - 128, (_NP - 128) // 16), 0)
    s2c = jax.lax.broadcasted_iota(jnp.int32, (_NP - 128, (_NP - 128) // 16), 1)
    S2 = jnp.where(s2r // 16 == s2c, (1 << (s2r % 16)).astype(f32), 0.0)

    def iou(rows, csl):
        ix1 = jnp.maximum(rows[:, 0:1], cx1[:, csl])
        iy1 = jnp.maximum(rows[:, 1:2], cy1[:, csl])
        ix2 = jnp.minimum(rows[:, 2:3], cx2[:, csl])
        iy2 = jnp.minimum(rows[:, 3:4], cy2[:, csl])
        inter = jnp.maximum(ix2 - ix1, 0.0) * jnp.maximum(iy2 - iy1, 0.0)
        return inter / (rows[:, 4:5] + car[:, csl] - inter + 1e-9)

    for base, ni in _TILES:
        rows = rT[base:base + ni, :]
        rs0 = rows[:, 6:7]
        dsl = slice(base, base + ni)
        qd = iou(rows, dsl)
        precd = ((rs0 > cs0[:, dsl])
                 | ((rs0 == cs0[:, dsl]) & (rl[:ni, :ni] < cl[:ni, :ni])))
        Md = ((qd > _T) & precd).astype(f32)
        Mp[base // 16:(base + ni) // 16, dsl] = jnp.dot(
            PW[:ni // 16, :ni], Md, preferred_element_type=f32).astype(jnp.int32)
        if base + ni == _NP:
            break
        usl = slice(base + ni, _NP)
        W = _NP - (base + ni)
        Of = (iou(rows, usl) > _T).astype(f32)
        Mu = Of * (rs0 >= cs0[:, usl]).astype(f32)
        Mp[base // 16:(base + ni) // 16, usl] = jnp.dot(
            PW[:ni // 16, :ni], Mu, preferred_element_type=f32).astype(jnp.int32)
        c6 = cS[6:7, usl]
        r5 = rows[:, 5:6]
        Mlt = Of * (c6 > r5).astype(f32)
        pkT = jnp.dot(Mlt, S2[:W, :W // 16], preferred_element_type=f32)
        Mp[(base + ni) // 16:_NW, dsl] = jnp.transpose(pkT, (1, 0)).astype(jnp.int32)

    jr = jax.lax.broadcasted_iota(jnp.int32, (_NP, 128), 0)
    wc = jax.lax.broadcasted_iota(jnp.int32, (_NP, 128), 1)
    S = jnp.where(jr // 16 == wc, (1 << (jr % 16)).astype(f32), 0.0)
    Mpv = Mp[...]

    def keep_of(kw):
        sup = jnp.max(Mpv & kw, axis=0, keepdims=True)
        return sup == 0

    def repack(kb):
        kwf = jnp.dot(kb.astype(f32), S, preferred_element_type=f32)
        return jnp.transpose(kwf, (1, 0))[:_NW, :].astype(jnp.int32)

    def cond(c):
        return c[1]

    def step(c):
        kw, _ = c
        nkw = repack(keep_of(kw))
        return nkw, jnp.any(nkw != kw)

    kw0 = jnp.full((_NW, 1), 0xFFFF, jnp.int32)
    kwf, _ = jax.lax.while_loop(cond, step, (kw0, jnp.bool_(True)))
    prob = keep_of(kwf).astype(f32) * cs0
    prob_ref[0, 0, :] = prob[0]


def kernel(x, im_info):
    B = x.shape[0]
    xp = x.reshape(B, 125, _POS)
    prob_pad, bb_pad = pl.pallas_call(
        _body,
        grid=(B,),
        in_specs=[
            pl.BlockSpec((1, 125, _POS), lambda i: (i, 0, 0)),
            pl.BlockSpec((1, 1, 2), lambda i: (i, 0, 0)),
        ],
        out_specs=[
            pl.BlockSpec((1, 1, _NP), lambda i: (i, 0, 0)),
            pl.BlockSpec((1, 4, _NP), lambda i: (i, 0, 0)),
        ],
        out_shape=[
            jax.ShapeDtypeStruct((B, 1, _NP), jnp.float32),
            jax.ShapeDtypeStruct((B, 4, _NP), jnp.float32),
        ],
        scratch_shapes=[
            pltpu.VMEM((8, _NP), jnp.float32),
            pltpu.VMEM((_NP, 8), jnp.float32),
            pltpu.VMEM((_NW, _NP), jnp.int32),
        ],
    )(xp, im_info.reshape(B, 1, 2))
    prob = prob_pad[:, 0, :_NC]
    bboxs = jnp.moveaxis(bb_pad[:, :, :_NC], 1, 2)
    return prob, bboxs
